# baseline (speedup 1.0000x reference)
import functools

import jax
import jax.numpy as jnp
from jax import lax
from jax.experimental import pallas as pl
from jax.experimental.pallas import tpu as pltpu
from jax.experimental.pallas import tpu_sc as plsc

_INFO = plsc.get_sparse_core_info()
_NC, _NS = _INFO.num_cores, _INFO.num_subcores
_NW = _NC * _NS
_CHAINS = (1, 6, 1)


def _make_gather(n_rows: int, row_len: int, chunk: int, row_off: int):
    assert n_rows % (_NW * chunk) == 0
    rows_per_w = n_rows // _NW
    n_chunks = rows_per_w // chunk
    assert n_chunks >= 2
    mesh = plsc.VectorSubcoreMesh(core_axis_name="c", subcore_axis_name="s")

    @functools.partial(
        pl.kernel,
        mesh=mesh,
        out_type=jax.ShapeDtypeStruct((n_rows, row_len), jnp.float32),
        scratch_types=[
            pltpu.VMEM((chunk,), jnp.int32),
            pltpu.VMEM((chunk,), jnp.int32),
            pltpu.VMEM((chunk, row_len), jnp.float32),
            pltpu.VMEM((chunk, row_len), jnp.float32),
            pltpu.SemaphoreType.DMA,
            pltpu.SemaphoreType.DMA,
            pltpu.SemaphoreType.DMA,
            pltpu.SemaphoreType.DMA,
        ],
    )
    def gather_kernel(x_hbm, idx_hbm, out_hbm,
                      idx0, idx1, rows0, rows1, gsem0, gsem1, wsem0, wsem1):
        wid = lax.axis_index("s") * _NC + lax.axis_index("c")
        base = wid * rows_per_w
        idxs, rows = [idx0, idx1], [rows0, rows1]
        gsems, wsems = [gsem0, gsem1], [wsem0, wsem1]

        pltpu.sync_copy(idx_hbm.at[pl.ds(row_off + base, chunk)], idxs[0])
        gathers = [pltpu.async_copy(x_hbm.at[idxs[0]], rows[0], gsems[0]), None]
        writes = [None, None]
        for i in range(n_chunks):
            cur, nxt = i % 2, (i + 1) % 2
            if i + 1 < n_chunks:
                off = row_off + base + (i + 1) * chunk
                pltpu.sync_copy(idx_hbm.at[pl.ds(off, chunk)], idxs[nxt])
                if writes[nxt] is not None:
                    writes[nxt].wait()
                gathers[nxt] = pltpu.async_copy(
                    x_hbm.at[idxs[nxt]], rows[nxt], gsems[nxt])
            gathers[cur].wait()
            writes[cur] = pltpu.async_copy(
                rows[cur], out_hbm.at[pl.ds(base + i * chunk, chunk)],
                wsems[cur])
        writes[0].wait()
        writes[1].wait()

    return gather_kernel


def _mm_body(x_ref, w_ref, o_ref):
    x = x_ref[0, 0].astype(jnp.bfloat16)
    w = w_ref[0].astype(jnp.bfloat16)
    o_ref[0, 0] = jnp.dot(x, w, preferred_element_type=jnp.float32)


def _mm_acc_body(x_ref, w_ref, y_prev_ref, o_ref):
    del y_prev_ref
    _mm_body(x_ref, w_ref, o_ref)


def _matmul_chain(xg, w, e_off, y_prev):
    ec, b, k, i = xg.shape
    e, _, j = w.shape
    y_shape = jax.ShapeDtypeStruct((b, e, k, j), jnp.float32)
    x_spec = pl.BlockSpec((1, 1, k, i), lambda ei, bi: (ei, bi, 0, 0))
    w_spec = pl.BlockSpec((1, i, j), lambda ei, bi: (ei + e_off, 0, 0))
    o_spec = pl.BlockSpec((1, 1, k, j), lambda ei, bi: (bi, ei + e_off, 0, 0))
    if y_prev is None:
        return pl.pallas_call(
            _mm_body,
            grid=(ec, b),
            in_specs=[x_spec, w_spec],
            out_specs=o_spec,
            out_shape=y_shape,
        )(xg, w)
    return pl.pallas_call(
        _mm_acc_body,
        grid=(ec, b),
        in_specs=[x_spec, w_spec,
                  pl.BlockSpec(memory_space=pltpu.MemorySpace.HBM)],
        out_specs=o_spec,
        out_shape=y_shape,
        input_output_aliases={2: 0},
    )(xg, w, y_prev)


def kernel(X, ind, W):
    B, T, I = X.shape
    _, E, K = ind.shape
    n_rows = B * E * K
    flat_idx = (
        ind.transpose(1, 0, 2)
        + (jnp.arange(B, dtype=jnp.int32) * T)[None, :, None]
    ).reshape(n_rows)
    x2d = X.reshape(B * T, I)

    y = None
    e_off = 0
    for e_cnt in _CHAINS:
        rows = e_cnt * B * K
        chunk = max(8, min(48, rows // (_NW * 2)))
        gather = _make_gather(rows, I, chunk, e_off * B * K)
        xg = gather(x2d, flat_idx)
        xg = xg.reshape(e_cnt, B, K, I)
        y = _matmul_chain(xg, W, e_off, y)
        e_off += e_cnt
    return y


# speedup vs baseline: 1.0438x; 1.0438x over previous
import functools

import jax
import jax.numpy as jnp
from jax import lax
from jax.experimental import pallas as pl
from jax.experimental.pallas import tpu as pltpu
from jax.experimental.pallas import tpu_sc as plsc

_INFO = plsc.get_sparse_core_info()
_NC, _NS = _INFO.num_cores, _INFO.num_subcores
_NW = _NC * _NS
_CHAINS = (5, 3)


def _make_gather(n_rows: int, row_len: int, chunk: int, row_off: int):
    assert n_rows % (_NW * chunk) == 0
    rows_per_w = n_rows // _NW
    n_chunks = rows_per_w // chunk
    assert n_chunks >= 2
    mesh = plsc.VectorSubcoreMesh(core_axis_name="c", subcore_axis_name="s")

    @functools.partial(
        pl.kernel,
        mesh=mesh,
        out_type=jax.ShapeDtypeStruct((n_rows, row_len), jnp.float32),
        scratch_types=[
            pltpu.VMEM((chunk,), jnp.int32),
            pltpu.VMEM((chunk,), jnp.int32),
            pltpu.VMEM((chunk, row_len), jnp.float32),
            pltpu.VMEM((chunk, row_len), jnp.float32),
            pltpu.SemaphoreType.DMA,
            pltpu.SemaphoreType.DMA,
            pltpu.SemaphoreType.DMA,
            pltpu.SemaphoreType.DMA,
        ],
    )
    def gather_kernel(x_hbm, idx_hbm, out_hbm,
                      idx0, idx1, rows0, rows1, gsem0, gsem1, wsem0, wsem1):
        wid = lax.axis_index("s") * _NC + lax.axis_index("c")
        base = wid * rows_per_w
        idxs, rows = [idx0, idx1], [rows0, rows1]
        gsems, wsems = [gsem0, gsem1], [wsem0, wsem1]

        pltpu.sync_copy(idx_hbm.at[pl.ds(row_off + base, chunk)], idxs[0])
        gathers = [pltpu.async_copy(x_hbm.at[idxs[0]], rows[0], gsems[0]), None]
        writes = [None, None]
        for i in range(n_chunks):
            cur, nxt = i % 2, (i + 1) % 2
            if i + 1 < n_chunks:
                off = row_off + base + (i + 1) * chunk
                pltpu.sync_copy(idx_hbm.at[pl.ds(off, chunk)], idxs[nxt])
                if writes[nxt] is not None:
                    writes[nxt].wait()
                gathers[nxt] = pltpu.async_copy(
                    x_hbm.at[idxs[nxt]], rows[nxt], gsems[nxt])
            gathers[cur].wait()
            writes[cur] = pltpu.async_copy(
                rows[cur], out_hbm.at[pl.ds(base + i * chunk, chunk)],
                wsems[cur])
        writes[0].wait()
        writes[1].wait()

    return gather_kernel


def _mm_body(x_ref, w_ref, o_ref):
    x = x_ref[0, 0].astype(jnp.bfloat16)
    w = w_ref[0].astype(jnp.bfloat16)
    o_ref[0, 0] = jnp.dot(x, w, preferred_element_type=jnp.float32)


def _mm_acc_body(x_ref, w_ref, y_prev_ref, o_ref):
    del y_prev_ref
    _mm_body(x_ref, w_ref, o_ref)


def _matmul_chain(xg, w, e_off, y_prev):
    ec, b, k, i = xg.shape
    e, _, j = w.shape
    y_shape = jax.ShapeDtypeStruct((b, e, k, j), jnp.float32)
    x_spec = pl.BlockSpec((1, 1, k, i), lambda ei, bi: (ei, bi, 0, 0))
    w_spec = pl.BlockSpec((1, i, j), lambda ei, bi: (ei + e_off, 0, 0))
    o_spec = pl.BlockSpec((1, 1, k, j), lambda ei, bi: (bi, ei + e_off, 0, 0))
    if y_prev is None:
        return pl.pallas_call(
            _mm_body,
            grid=(ec, b),
            in_specs=[x_spec, w_spec],
            out_specs=o_spec,
            out_shape=y_shape,
        )(xg, w)
    return pl.pallas_call(
        _mm_acc_body,
        grid=(ec, b),
        in_specs=[x_spec, w_spec,
                  pl.BlockSpec(memory_space=pltpu.MemorySpace.HBM)],
        out_specs=o_spec,
        out_shape=y_shape,
        input_output_aliases={2: 0},
    )(xg, w, y_prev)


def kernel(X, ind, W):
    B, T, I = X.shape
    _, E, K = ind.shape
    n_rows = B * E * K
    flat_idx = (
        ind.transpose(1, 0, 2)
        + (jnp.arange(B, dtype=jnp.int32) * T)[None, :, None]
    ).reshape(n_rows)
    x2d = X.reshape(B * T, I)

    y = None
    e_off = 0
    for e_cnt in _CHAINS:
        rows = e_cnt * B * K
        rows_per_w = rows // _NW
        chunk = next(c for c in range(min(48, rows_per_w // 2), 7, -1)
                     if c % 8 == 0 and rows_per_w % c == 0)
        gather = _make_gather(rows, I, chunk, e_off * B * K)
        xg = gather(x2d, flat_idx)
        xg = xg.reshape(e_cnt, B, K, I)
        y = _matmul_chain(xg, W, e_off, y)
        e_off += e_cnt
    return y
